# baseline (speedup 1.0000x reference)
import functools

import jax
import jax.numpy as jnp
from jax import lax
from jax.experimental import pallas as pl
from jax.experimental.pallas import tpu as pltpu
from jax.experimental.pallas import tpu_sc as plsc

NC = 2
NS = 16
L = 16


def _rsqrt_f32(x):
    xi = plsc.bitcast(x, jnp.int32)
    y = plsc.bitcast(jnp.int32(0x5F3759DF) - (xi >> 1), jnp.float32)
    for _ in range(2):
        y = y * (1.5 - 0.5 * x * y * y)
    return y


@functools.partial(jax.jit, static_argnames=("n_nodes", "n_edges"))
def _norm_sc(ref_a, ref_b, ones, zeros, *, n_nodes, n_edges):
    eo = n_edges // (NC * NS)

    mesh = plsc.VectorSubcoreMesh(core_axis_name="c", subcore_axis_name="s")

    @functools.partial(
        pl.kernel,
        out_type=(
            jax.ShapeDtypeStruct((n_edges,), jnp.float32),
            jax.ShapeDtypeStruct((NC, n_nodes), jnp.float32),
        ),
        mesh=mesh,
        compiler_params=pltpu.CompilerParams(needs_layout_passes=False),
        scratch_types=[
            pltpu.VMEM_SHARED((n_nodes,), jnp.float32),
            pltpu.VMEM((eo,), jnp.int32),
            pltpu.VMEM((eo,), jnp.int32),
            pltpu.VMEM((eo,), jnp.int32),
            pltpu.VMEM((eo,), jnp.float32),
            pltpu.VMEM((n_nodes,), jnp.float32),
            pltpu.VMEM((n_nodes,), jnp.float32),
            pltpu.VMEM((eo,), jnp.float32),
            pltpu.SemaphoreType.DMA,
            pltpu.SemaphoreType.DMA,
            pltpu.SemaphoreType.DMA,
            pltpu.SemaphoreType.DMA,
            pltpu.SemaphoreType.DMA,
            pltpu.SemaphoreType.DMA,
            pltpu.SemaphoreType.DMA,
            pltpu.SemaphoreType.REGULAR,
        ],
    )
    def norm_kernel(a_hbm, b_hbm, ones_hbm, zeros_hbm, out_hbm, pub_hbm,
                    deg_sh, i1_v, i2_v, io_v, ones_v, down_v, doth_v, out_v,
                    sem_i1, sem_i2, sem_io, sem_ones, sem_sc,
                    sem_down, sem_doth, xsem):
        c = lax.axis_index("c")
        s = lax.axis_index("s")
        base_h = s * (2 * eo)
        base_o = base_h + c * eo

        cp_ones = pltpu.async_copy(ones_hbm, ones_v, sem_ones)

        def hist(src_hbm, oth_hbm):
            cp_i1 = pltpu.async_copy(
                src_hbm.at[pl.ds(base_h, eo)], i1_v, sem_i1)
            cp_i2 = pltpu.async_copy(
                src_hbm.at[pl.ds(base_h + eo, eo)], i2_v, sem_i2)
            cp_io = pltpu.async_copy(
                oth_hbm.at[pl.ds(base_o, eo)], io_v, sem_io)

            @pl.when(s == 0)
            def _():
                pltpu.sync_copy(zeros_hbm, deg_sh)

            plsc.subcore_barrier()

            cp_i1.wait()
            cp_ones.wait()
            sc1 = pltpu.async_copy(ones_v, deg_sh.at[i1_v], sem_sc, add=True)
            cp_i2.wait()
            sc2 = pltpu.async_copy(ones_v, deg_sh.at[i2_v], sem_sc, add=True)
            sc1.wait()
            sc2.wait()
            plsc.subcore_barrier()

            @pl.when(s == 0)
            def _():
                pltpu.sync_copy(deg_sh, pub_hbm.at[c])
                pl.semaphore_signal(xsem, 1, core_index=1 - c)
                pl.semaphore_wait(xsem, 1)

            plsc.subcore_barrier()
            cp_io.wait()

        @pl.when(c == 0)
        def _():
            hist(a_hbm, b_hbm)

        @pl.when(c == 1)
        def _():
            hist(b_hbm, a_hbm)

        cp_down = pltpu.async_copy(deg_sh, down_v, sem_down)
        cp_doth = pltpu.async_copy(pub_hbm.at[1 - c], doth_v, sem_doth)
        cp_down.wait()
        cp_doth.wait()

        def emit(own_idx_v):
            @plsc.parallel_loop(0, eo, step=L, unroll=16)
            def _(i):
                sl = pl.ds(i, L)
                down = plsc.load_gather(down_v, [own_idx_v[sl]])
                doth = plsc.load_gather(doth_v, [io_v[sl]])
                out_v[sl] = _rsqrt_f32(down * doth)

        @pl.when(c == 0)
        def _():
            emit(i1_v)

        @pl.when(c == 1)
        def _():
            emit(i2_v)

        pltpu.sync_copy(out_v, out_hbm.at[pl.ds(base_o, eo)])

    return norm_kernel(ref_a, ref_b, ones, zeros)[0]


def kernel(X, ref_a, ref_b):
    n_nodes = X.shape[0]
    n_edges = ref_a.shape[0]
    ones = jnp.ones((n_edges // (NC * NS),), jnp.float32)
    zeros = jnp.zeros((n_nodes,), jnp.float32)
    return _norm_sc(
        ref_a.astype(jnp.int32),
        ref_b.astype(jnp.int32),
        ones,
        zeros,
        n_nodes=n_nodes,
        n_edges=n_edges,
    )


# speedup vs baseline: 1.0089x; 1.0089x over previous
import functools

import jax
import jax.numpy as jnp
from jax import lax
from jax.experimental import pallas as pl
from jax.experimental.pallas import tpu as pltpu
from jax.experimental.pallas import tpu_sc as plsc

NC = 2
NS = 16
L = 16


def _rsqrt_f32(x):
    xi = plsc.bitcast(x, jnp.int32)
    y = plsc.bitcast(jnp.int32(0x5F3759DF) - (xi >> 1), jnp.float32)
    for _ in range(2):
        y = y * (1.5 - 0.5 * x * y * y)
    return y


@functools.partial(jax.jit, static_argnames=("n_nodes", "n_edges"))
def _norm_sc(ref_a, ref_b, ones, zeros, *, n_nodes, n_edges):
    eo = n_edges // (NC * NS)

    mesh = plsc.VectorSubcoreMesh(core_axis_name="c", subcore_axis_name="s")

    @functools.partial(
        pl.kernel,
        out_type=(
            jax.ShapeDtypeStruct((n_edges,), jnp.float32),
            jax.ShapeDtypeStruct((NC, n_nodes), jnp.float32),
        ),
        mesh=mesh,
        compiler_params=pltpu.CompilerParams(needs_layout_passes=False),
        scratch_types=[
            pltpu.VMEM_SHARED((n_nodes,), jnp.float32),
            pltpu.VMEM((eo,), jnp.int32),
            pltpu.VMEM((eo,), jnp.int32),
            pltpu.VMEM((eo,), jnp.int32),
            pltpu.VMEM((eo,), jnp.float32),
            pltpu.VMEM((n_nodes,), jnp.float32),
            pltpu.VMEM((n_nodes,), jnp.float32),
            pltpu.VMEM((eo,), jnp.float32),
            pltpu.SemaphoreType.DMA,
            pltpu.SemaphoreType.DMA,
            pltpu.SemaphoreType.DMA,
            pltpu.SemaphoreType.DMA,
            pltpu.SemaphoreType.DMA,
            pltpu.SemaphoreType.DMA,
            pltpu.SemaphoreType.DMA,
            pltpu.SemaphoreType.REGULAR,
        ],
    )
    def norm_kernel(a_hbm, b_hbm, ones_hbm, zeros_hbm, out_hbm, pub_hbm,
                    deg_sh, i1_v, i2_v, io_v, ones_v, down_v, doth_v, out_v,
                    sem_i1, sem_i2, sem_io, sem_ones, sem_sc,
                    sem_down, sem_doth, xsem):
        c = lax.axis_index("c")
        s = lax.axis_index("s")
        base_h = s * (2 * eo)
        base_o = base_h + c * eo

        cp_ones = pltpu.async_copy(ones_hbm, ones_v, sem_ones)

        def hist(src_hbm, oth_hbm):
            cp_i1 = pltpu.async_copy(
                src_hbm.at[pl.ds(base_h, eo)], i1_v, sem_i1)
            cp_i2 = pltpu.async_copy(
                src_hbm.at[pl.ds(base_h + eo, eo)], i2_v, sem_i2)
            cp_io = pltpu.async_copy(
                oth_hbm.at[pl.ds(base_o, eo)], io_v, sem_io)

            @pl.when(s == 0)
            def _():
                pltpu.sync_copy(zeros_hbm, deg_sh)

            plsc.subcore_barrier()

            cp_i1.wait()
            cp_ones.wait()
            sc1 = pltpu.async_copy(ones_v, deg_sh.at[i1_v], sem_sc, add=True)
            cp_i2.wait()
            sc2 = pltpu.async_copy(ones_v, deg_sh.at[i2_v], sem_sc, add=True)
            sc1.wait()
            sc2.wait()
            plsc.subcore_barrier()

            cp_down = pltpu.async_copy(deg_sh, down_v, sem_down)

            @pl.when(s == 0)
            def _():
                pltpu.sync_copy(deg_sh, pub_hbm.at[c])
                pl.semaphore_signal(xsem, 1, core_index=1 - c)
                pl.semaphore_wait(xsem, 1)

            plsc.subcore_barrier()
            cp_io.wait()
            cp_down.wait()

        @pl.when(c == 0)
        def _():
            hist(a_hbm, b_hbm)

        @pl.when(c == 1)
        def _():
            hist(b_hbm, a_hbm)

        cp_doth = pltpu.async_copy(pub_hbm.at[1 - c], doth_v, sem_doth)
        cp_doth.wait()

        def emit(own_idx_v):
            @plsc.parallel_loop(0, eo, step=L, unroll=8)
            def _(i):
                sl = pl.ds(i, L)
                down = plsc.load_gather(down_v, [own_idx_v[sl]])
                doth = plsc.load_gather(doth_v, [io_v[sl]])
                out_v[sl] = _rsqrt_f32(down * doth)

        @pl.when(c == 0)
        def _():
            emit(i1_v)

        @pl.when(c == 1)
        def _():
            emit(i2_v)

        pltpu.sync_copy(out_v, out_hbm.at[pl.ds(base_o, eo)])

    return norm_kernel(ref_a, ref_b, ones, zeros)[0]


def kernel(X, ref_a, ref_b):
    n_nodes = X.shape[0]
    n_edges = ref_a.shape[0]
    ones = jnp.ones((n_edges // (NC * NS),), jnp.float32)
    zeros = jnp.zeros((n_nodes,), jnp.float32)
    return _norm_sc(
        ref_a.astype(jnp.int32),
        ref_b.astype(jnp.int32),
        ones,
        zeros,
        n_nodes=n_nodes,
        n_edges=n_edges,
    )
